# baseline (speedup 1.0000x reference)
import functools

import jax
import jax.numpy as jnp
from jax import lax
from jax.experimental import pallas as pl
from jax.experimental.pallas import tpu as pltpu
from jax.experimental.pallas import tpu_sc as plsc

N_NODES = 10000
N_EDGES = 320000
DIM = 128

NUM_CORES = 2
NUM_SUBCORES = 16
NUM_WORKERS = NUM_CORES * NUM_SUBCORES

CHUNK = 128
CHUNKS_C0 = 104
CHUNKS_C1 = 56
CHUNKS_MAX = max(CHUNKS_C0, CHUNKS_C1)
NUM_CHUNKS = NUM_SUBCORES * (CHUNKS_C0 + CHUNKS_C1)
EDGES_PADDED = NUM_CHUNKS * CHUNK
W_ROWS = NUM_CHUNKS + 8

ROWS_PER_TILE = 624
REM_BASE = NUM_SUBCORES * ROWS_PER_TILE
REM_ROWS = N_NODES - REM_BASE


def _take16(vec, idx16):
    return lax.gather(
        vec, idx16[:, None],
        lax.GatherDimensionNumbers(offset_dims=(), collapsed_slice_dims=(0,),
                                   start_index_map=(0,)),
        slice_sizes=(1,),
        mode=lax.GatherScatterMode.PROMISE_IN_BOUNDS)

def _sc_body(emb_hbm, meta_hbm, w_hbm, out_hbm,
             meta_v, rows_v, w_all, acc,
             sem_m0, sem_m1, sem_g0, sem_g1):
    cid = lax.axis_index("c")
    sid = lax.axis_index("s")
    c_count = jnp.where(cid == 0, CHUNKS_C0, CHUNKS_C1)
    chunk0 = cid * (NUM_SUBCORES * CHUNKS_C0) + sid * c_count
    sem_m = (sem_m0, sem_m1)
    sem_g = (sem_g0, sem_g1)

    zeros16 = jnp.zeros((16,), jnp.float32)

    def zero_row(i, _):
        for q in range(8):
            rows_v[0, i, pl.ds(q * 16, 16)] = zeros16
        return _

    lax.fori_loop(0, CHUNK, zero_row, None)
    base = sid * ROWS_PER_TILE
    off = 0
    for sz in (128, 128, 128, 128, 112):
        pltpu.sync_copy(rows_v.at[0, pl.ds(0, sz)],
                        acc.at[pl.ds(base + off, sz)])
        off += sz

    @pl.when(sid == 0)
    def _():
        pltpu.sync_copy(rows_v.at[0, pl.ds(0, REM_ROWS)],
                        acc.at[pl.ds(REM_BASE, REM_ROWS)])

    plsc.subcore_barrier()

    pltpu.sync_copy(w_hbm.at[pl.ds(chunk0, CHUNKS_MAX)], w_all)

    def start_meta(c, b):
        pltpu.async_copy(meta_hbm.at[chunk0 + c], meta_v.at[b], sem_m[b])

    def wait_meta(c, b):
        pltpu.make_async_copy(meta_hbm.at[chunk0 + c], meta_v.at[b],
                              sem_m[b]).wait()

    def start_gather(b):
        pltpu.async_copy(emb_hbm.at[meta_v.at[b, 0]], rows_v.at[b], sem_g[b])

    def wait_gather(b):
        pltpu.make_async_copy(emb_hbm.at[meta_v.at[b, 0]], rows_v.at[b],
                              sem_g[b]).wait()

    def process(c, b):
        for q in range(8):
            wv = w_all[c, pl.ds(q * 16, 16)]

            def mul_body(l, _):
                e = q * 16 + l
                wl = _take16(wv, jnp.full((16,), l, jnp.int32))
                for k in range(8):
                    rows_v[b, e, pl.ds(k * 16, 16)] = (
                        rows_v[b, e, pl.ds(k * 16, 16)] * wl)
                return _

            lax.fori_loop(0, 16, mul_body, None)
        pltpu.sync_copy(rows_v.at[b], acc.at[meta_v.at[b, 1]], add=True)

    start_meta(0, 0)
    start_meta(1, 1)
    wait_meta(0, 0)
    start_gather(0)

    def chunk_pair(g2, _):
        c = g2 * 2
        for b in (0, 1):
            wait_meta(c + 1, 1 - b)
            start_gather(1 - b)
            wait_gather(b)
            process(c, b)
            start_meta(c + 2, b)
            c = c + 1
        return _

    lax.fori_loop(0, (c_count - 2) // 2, chunk_pair, None)

    wait_meta(c_count - 1, 1)
    start_gather(1)
    wait_gather(0)
    process(c_count - 2, 0)
    wait_gather(1)
    process(c_count - 1, 1)

    plsc.subcore_barrier()
    pltpu.sync_copy(acc.at[pl.ds(base, ROWS_PER_TILE)],
                    out_hbm.at[cid, pl.ds(base, ROWS_PER_TILE)])

    @pl.when(sid == 0)
    def _():
        pltpu.sync_copy(acc.at[pl.ds(REM_BASE, REM_ROWS)],
                        out_hbm.at[cid, pl.ds(REM_BASE, REM_ROWS)])


_sc_kernel = functools.partial(
    pl.kernel,
    out_type=jax.ShapeDtypeStruct((NUM_CORES, N_NODES, DIM), jnp.float32),
    mesh=plsc.VectorSubcoreMesh(core_axis_name="c", subcore_axis_name="s"),
    scratch_types=[
        pltpu.VMEM((2, 2, 128), jnp.int32),
        pltpu.VMEM((2, CHUNK, DIM), jnp.float32),
        pltpu.VMEM((CHUNKS_MAX, 128), jnp.float32),
        pltpu.VMEM_SHARED((N_NODES, DIM), jnp.float32),
        pltpu.SemaphoreType.DMA,
        pltpu.SemaphoreType.DMA,
        pltpu.SemaphoreType.DMA,
        pltpu.SemaphoreType.DMA,
    ],
)(_sc_body)


def _tc_body(e_ref, p0_ref, p1_ref, w1_ref, w2_ref, b1_ref, b2_ref, out_ref):
    e = e_ref[...]
    nb = p0_ref[...] + p1_ref[...]
    t = nb + nb * e
    out = (jnp.dot(e, w1_ref[...], preferred_element_type=jnp.float32)
           + jnp.dot(t, w2_ref[...], preferred_element_type=jnp.float32)
           + b1_ref[...] + b2_ref[...])
    out = jnp.where(out >= 0, out, 0.2 * out)
    nrm = jnp.sqrt(jnp.sum(out * out, axis=1, keepdims=True))
    out_ref[...] = out / jnp.maximum(nrm, 1e-12)


ROW_BLOCK = 1000


def _tc_epilogue(emb, p0, p1, W1, W2, b1, b2):
    grid = (N_NODES // ROW_BLOCK,)
    row_spec = pl.BlockSpec((ROW_BLOCK, DIM), lambda i: (i, 0))
    full_spec = pl.BlockSpec((DIM, DIM), lambda i: (0, 0))
    bias_spec = pl.BlockSpec((1, DIM), lambda i: (0, 0))
    return pl.pallas_call(
        _tc_body,
        grid=grid,
        in_specs=[row_spec, row_spec, row_spec, full_spec, full_spec,
                  bias_spec, bias_spec],
        out_specs=row_spec,
        out_shape=jax.ShapeDtypeStruct((N_NODES, DIM), jnp.float32),
    )(emb, p0, p1, W1, W2, b1.reshape(1, DIM), b2.reshape(1, DIM))


@jax.jit
def kernel(embeddings, edge_index, edge_weight, W1, b1, W2, b2):
    src = edge_index[0].astype(jnp.int32)
    dst = edge_index[1].astype(jnp.int32)
    w = edge_weight.astype(jnp.float32)
    pad = EDGES_PADDED - N_EDGES
    src = jnp.pad(src, (0, pad)).reshape(NUM_CHUNKS, 128)
    dst = jnp.pad(dst, (0, pad)).reshape(NUM_CHUNKS, 128)
    w = jnp.pad(w, (0, pad + (W_ROWS - NUM_CHUNKS) * 128)).reshape(W_ROWS, 128)
    meta = jnp.stack([src, dst], axis=1)

    partials = _sc_kernel(embeddings, meta, w)
    return _tc_epilogue(embeddings, partials[0], partials[1],
                        W1, W2, b1, b2)


# speedup vs baseline: 1.0467x; 1.0467x over previous
import functools

import jax
import jax.numpy as jnp
from jax import lax
from jax.experimental import pallas as pl
from jax.experimental.pallas import tpu as pltpu
from jax.experimental.pallas import tpu_sc as plsc

N_NODES = 10000
N_EDGES = 320000
DIM = 128

NUM_CORES = 2
NUM_SUBCORES = 16
NUM_WORKERS = NUM_CORES * NUM_SUBCORES

CHUNK = 128
CHUNKS_C0 = 136
CHUNKS_C1 = 24
CHUNKS_MAX = max(CHUNKS_C0, CHUNKS_C1)
NUM_CHUNKS = NUM_SUBCORES * (CHUNKS_C0 + CHUNKS_C1)
EDGES_PADDED = NUM_CHUNKS * CHUNK
W_ROWS = NUM_CHUNKS + 8

ROWS_PER_TILE = 624
REM_BASE = NUM_SUBCORES * ROWS_PER_TILE
REM_ROWS = N_NODES - REM_BASE


def _take16(vec, idx16):
    return lax.gather(
        vec, idx16[:, None],
        lax.GatherDimensionNumbers(offset_dims=(), collapsed_slice_dims=(0,),
                                   start_index_map=(0,)),
        slice_sizes=(1,),
        mode=lax.GatherScatterMode.PROMISE_IN_BOUNDS)

def _sc_body(emb_hbm, meta_hbm, w_hbm, out_hbm,
             meta_v, rows_v, w_all, acc,
             sem_m0, sem_m1, sem_g0, sem_g1):
    cid = lax.axis_index("c")
    sid = lax.axis_index("s")
    c_count = jnp.where(cid == 0, CHUNKS_C0, CHUNKS_C1)
    chunk0 = cid * (NUM_SUBCORES * CHUNKS_C0) + sid * c_count
    sem_m = (sem_m0, sem_m1)
    sem_g = (sem_g0, sem_g1)

    zeros16 = jnp.zeros((16,), jnp.float32)

    def zero_row(i, _):
        for q in range(8):
            rows_v[0, i, pl.ds(q * 16, 16)] = zeros16
        return _

    lax.fori_loop(0, CHUNK, zero_row, None)
    base = sid * ROWS_PER_TILE
    off = 0
    for sz in (128, 128, 128, 128, 112):
        pltpu.sync_copy(rows_v.at[0, pl.ds(0, sz)],
                        acc.at[pl.ds(base + off, sz)])
        off += sz

    @pl.when(sid == 0)
    def _():
        pltpu.sync_copy(rows_v.at[0, pl.ds(0, REM_ROWS)],
                        acc.at[pl.ds(REM_BASE, REM_ROWS)])

    plsc.subcore_barrier()

    pltpu.sync_copy(w_hbm.at[pl.ds(chunk0, CHUNKS_MAX)], w_all)

    def start_meta(c, b):
        pltpu.async_copy(meta_hbm.at[chunk0 + c], meta_v.at[b], sem_m[b])

    def wait_meta(c, b):
        pltpu.make_async_copy(meta_hbm.at[chunk0 + c], meta_v.at[b],
                              sem_m[b]).wait()

    def start_gather(b):
        pltpu.async_copy(emb_hbm.at[meta_v.at[b, 0]], rows_v.at[b], sem_g[b])

    def wait_gather(b):
        pltpu.make_async_copy(emb_hbm.at[meta_v.at[b, 0]], rows_v.at[b],
                              sem_g[b]).wait()

    def process(c, b):
        for q in range(8):
            wv = w_all[c, pl.ds(q * 16, 16)]

            def mul_body(l, _):
                e = q * 16 + l
                wl = _take16(wv, jnp.full((16,), l, jnp.int32))
                for k in range(8):
                    rows_v[b, e, pl.ds(k * 16, 16)] = (
                        rows_v[b, e, pl.ds(k * 16, 16)] * wl)
                return _

            lax.fori_loop(0, 16, mul_body, None)
        pltpu.sync_copy(rows_v.at[b], acc.at[meta_v.at[b, 1]], add=True)

    start_meta(0, 0)
    start_meta(1, 1)
    wait_meta(0, 0)
    start_gather(0)

    def chunk_pair(g2, _):
        c = g2 * 2
        for b in (0, 1):
            wait_meta(c + 1, 1 - b)
            start_gather(1 - b)
            wait_gather(b)
            process(c, b)
            start_meta(c + 2, b)
            c = c + 1
        return _

    lax.fori_loop(0, (c_count - 2) // 2, chunk_pair, None)

    wait_meta(c_count - 1, 1)
    start_gather(1)
    wait_gather(0)
    process(c_count - 2, 0)
    wait_gather(1)
    process(c_count - 1, 1)

    plsc.subcore_barrier()
    pltpu.sync_copy(acc.at[pl.ds(base, ROWS_PER_TILE)],
                    out_hbm.at[cid, pl.ds(base, ROWS_PER_TILE)])

    @pl.when(sid == 0)
    def _():
        pltpu.sync_copy(acc.at[pl.ds(REM_BASE, REM_ROWS)],
                        out_hbm.at[cid, pl.ds(REM_BASE, REM_ROWS)])


_sc_kernel = functools.partial(
    pl.kernel,
    out_type=jax.ShapeDtypeStruct((NUM_CORES, N_NODES, DIM), jnp.float32),
    mesh=plsc.VectorSubcoreMesh(core_axis_name="c", subcore_axis_name="s"),
    scratch_types=[
        pltpu.VMEM((2, 2, 128), jnp.int32),
        pltpu.VMEM((2, CHUNK, DIM), jnp.float32),
        pltpu.VMEM((CHUNKS_MAX, 128), jnp.float32),
        pltpu.VMEM_SHARED((N_NODES, DIM), jnp.float32),
        pltpu.SemaphoreType.DMA,
        pltpu.SemaphoreType.DMA,
        pltpu.SemaphoreType.DMA,
        pltpu.SemaphoreType.DMA,
    ],
)(_sc_body)


def _tc_body(e_ref, p0_ref, p1_ref, w1_ref, w2_ref, b1_ref, b2_ref, out_ref):
    e = e_ref[...]
    nb = p0_ref[...] + p1_ref[...]
    t = nb + nb * e
    out = (jnp.dot(e, w1_ref[...], preferred_element_type=jnp.float32)
           + jnp.dot(t, w2_ref[...], preferred_element_type=jnp.float32)
           + b1_ref[...] + b2_ref[...])
    out = jnp.where(out >= 0, out, 0.2 * out)
    nrm = jnp.sqrt(jnp.sum(out * out, axis=1, keepdims=True))
    out_ref[...] = out / jnp.maximum(nrm, 1e-12)


ROW_BLOCK = 1000


def _tc_epilogue(emb, p0, p1, W1, W2, b1, b2):
    grid = (N_NODES // ROW_BLOCK,)
    row_spec = pl.BlockSpec((ROW_BLOCK, DIM), lambda i: (i, 0))
    full_spec = pl.BlockSpec((DIM, DIM), lambda i: (0, 0))
    bias_spec = pl.BlockSpec((1, DIM), lambda i: (0, 0))
    return pl.pallas_call(
        _tc_body,
        grid=grid,
        in_specs=[row_spec, row_spec, row_spec, full_spec, full_spec,
                  bias_spec, bias_spec],
        out_specs=row_spec,
        out_shape=jax.ShapeDtypeStruct((N_NODES, DIM), jnp.float32),
    )(emb, p0, p1, W1, W2, b1.reshape(1, DIM), b2.reshape(1, DIM))


@jax.jit
def kernel(embeddings, edge_index, edge_weight, W1, b1, W2, b2):
    src = edge_index[0].astype(jnp.int32)
    dst = edge_index[1].astype(jnp.int32)
    w = edge_weight.astype(jnp.float32)
    pad = EDGES_PADDED - N_EDGES
    src = jnp.pad(src, (0, pad)).reshape(NUM_CHUNKS, 128)
    dst = jnp.pad(dst, (0, pad)).reshape(NUM_CHUNKS, 128)
    w = jnp.pad(w, (0, pad + (W_ROWS - NUM_CHUNKS) * 128)).reshape(W_ROWS, 128)
    meta = jnp.stack([src, dst], axis=1)

    partials = _sc_kernel(embeddings, meta, w)
    return _tc_epilogue(embeddings, partials[0], partials[1],
                        W1, W2, b1, b2)
